# baseline (speedup 1.0000x reference)
import functools

import jax
import jax.numpy as jnp
from jax import lax
from jax.experimental import pallas as pl
from jax.experimental.pallas import tpu as pltpu
from jax.experimental.pallas import tpu_sc as plsc

KSEL = 256

_NC, _NS = 2, 16
_NW = _NC * _NS

_ERF_ALPHA = (-2.72614225801306e-10, 2.77068142495902e-08,
              -2.10102402082508e-06, -5.69250639462346e-05,
              -7.34990630326855e-04, -2.95459980854025e-03,
              -1.60960333262415e-02)
_ERF_BETA = (-1.45660718464996e-05, -2.13374055278905e-04,
             -1.68282697438203e-03, -7.37332916720468e-03,
             -1.42647390514189e-02)


def _erf(z):
    z = jnp.clip(z, -4.0, 4.0)
    z2 = z * z
    alpha = jnp.float32(_ERF_ALPHA[0])
    for c in _ERF_ALPHA[1:]:
        alpha = alpha * z2 + jnp.float32(c)
    beta = jnp.float32(_ERF_BETA[0])
    for c in _ERF_BETA[1:]:
        beta = beta * z2 + jnp.float32(c)
    return z * alpha / beta


def _gelu_exact(x):
    return 0.5 * x * (1.0 + _erf(x * jnp.float32(0.7071067811865476)))


def _make_sc_colgather_body(off):
    def body(w_ref, idx_ref, out_ref, idx_v, row_a, row_b, out_v, sem_a,
             sem_b, osem):
        rows = out_v.shape[0]
        k_in = out_v.shape[1]
        wid = lax.axis_index("s") * _NC + lax.axis_index("c")
        p0 = wid * rows

        ck = row_a.shape[0]
        pltpu.sync_copy(idx_ref.at[pl.ds(off, k_in)], idx_v)
        pltpu.async_copy(w_ref.at[pl.ds(p0, ck)], row_a, sem_a)

        def extract(buf, r0):
            for g in range(k_in // 16):
                cvec = idx_v[pl.ds(g * 16, 16)]
                for rr in range(ck):
                    rvec = jnp.full((16,), rr, jnp.int32)
                    vals = plsc.load_gather(buf, [rvec, cvec])
                    out_v[r0 + rr, pl.ds(g * 16, 16)] = vals

        n_pairs = rows // (2 * ck)

        def pair_body(cp, carry):
            r0 = 2 * ck * cp
            pltpu.async_copy(w_ref.at[pl.ds(p0 + r0 + ck, ck)], row_b, sem_b)
            pltpu.make_async_copy(w_ref.at[pl.ds(p0, ck)], row_a, sem_a).wait()
            extract(row_a, r0)
            pltpu.async_copy(out_v.at[pl.ds(r0, ck)],
                             out_ref.at[pl.ds(p0 + r0, ck)], osem)

            @pl.when(cp + 1 < n_pairs)
            def _():
                pltpu.async_copy(w_ref.at[pl.ds(p0 + r0 + 2 * ck, ck)], row_a,
                                 sem_a)
            pltpu.make_async_copy(w_ref.at[pl.ds(p0, ck)], row_b, sem_b).wait()
            extract(row_b, r0 + ck)
            pltpu.async_copy(out_v.at[pl.ds(r0 + ck, ck)],
                             out_ref.at[pl.ds(p0 + r0 + ck, ck)], osem)
            return carry

        lax.fori_loop(0, n_pairs, pair_body, jnp.int32(0))
        for _ in range(2 * n_pairs):
            pltpu.make_async_copy(out_v.at[pl.ds(0, ck)],
                                  out_ref.at[pl.ds(p0, ck)], osem).wait()

    return body


def _sc_gather_cols(w, idx_full, k_in, off):
    n_process, n_input = w.shape
    rows = n_process // _NW
    mesh = plsc.VectorSubcoreMesh(core_axis_name="c", subcore_axis_name="s")
    return pl.kernel(
        _make_sc_colgather_body(off),
        out_type=jax.ShapeDtypeStruct((n_process, k_in), w.dtype),
        mesh=mesh,
        compiler_params=pltpu.CompilerParams(needs_layout_passes=False),
        scratch_types=[
            pltpu.VMEM((k_in,), jnp.int32),
            pltpu.VMEM((8, n_input), w.dtype),
            pltpu.VMEM((8, n_input), w.dtype),
            pltpu.VMEM((rows, k_in), w.dtype),
            pltpu.SemaphoreType.DMA,
            pltpu.SemaphoreType.DMA,
            pltpu.SemaphoreType.DMA,
        ],
    )(w, idx_full)


def _topk_mask(s):
    n = s.shape[1]
    si = jax.lax.bitcast_convert_type(s, jnp.int32)
    keys = jnp.where(si >= 0, si, si ^ jnp.int32(0x7FFFFFFF))
    sign = jnp.int32(-2147483648)

    def tbody(i, p):
        bit = jnp.left_shift(jnp.int32(1), jnp.int32(31) - i)
        cand = p | bit
        cnt = jnp.sum(jnp.where(keys >= (cand ^ sign), jnp.int32(1), jnp.int32(0)))
        return jnp.where(cnt >= KSEL, cand, p)

    p = jax.lax.fori_loop(0, 32, tbody, jnp.int32(0))
    thr = p ^ sign

    gt = keys > thr
    eq = keys == thr
    deficit = KSEL - jnp.sum(jnp.where(gt, jnp.int32(1), jnp.int32(0)))
    pidx = jax.lax.broadcasted_iota(jnp.int32, (1, n), 1)

    def mbody(i, lohi):
        lo, hi = lohi
        mid = (lo + hi) // 2
        cnt = jnp.sum(jnp.where(eq & (pidx <= mid), jnp.int32(1), jnp.int32(0)))
        ok = cnt >= deficit
        return jnp.where(ok, lo, mid + 1), jnp.where(ok, mid, hi)

    lo, _ = jax.lax.fori_loop(0, 11, mbody, (jnp.int32(0), jnp.int32(n - 1)))
    return (gt | (eq & (pidx <= lo))).astype(jnp.float32)


_PT = 16
_ST = 4


def _fused_body(a_ref, w_ref, p_ref, *rest):
    out_ref, pa_scr, sc_scr, mask_scr = rest[-4:]
    i = pl.program_id(1)
    s_full, pblk = pa_scr.shape[1], pa_scr.shape[2]
    sblk = out_ref.shape[1]

    @pl.when(i < _PT)
    def _mm1():
        acts = lax.dot_general(a_ref[0], w_ref[0],
                               (((1,), (1,)), ((), ())),
                               preferred_element_type=jnp.float32)
        pa = _gelu_exact(acts)
        pa_scr[i] = pa
        sc_scr[i] = jnp.sum(pa, axis=0, keepdims=True)

    @pl.when(i == _PT)
    def _mask():
        s = jnp.concatenate([sc_scr[j] for j in range(_PT)], axis=-1)
        mask = _topk_mask(s)
        for j in range(_PT):
            mask_scr[j] = mask[:, j * pblk:(j + 1) * pblk]

    @pl.when(i >= _PT)
    def _mm2():
        st = i - _PT
        acc = jnp.zeros((sblk, out_ref.shape[2]), jnp.float32)
        for j in range(_PT):
            pa = pa_scr[j, pl.ds(st * sblk, sblk), :] * mask_scr[j]
            acc += jnp.dot(pa.astype(jnp.bfloat16),
                           p_ref[pl.ds(j * pblk, pblk), :],
                           preferred_element_type=jnp.float32)
        out_ref[0] = acc


def kernel(selected_activations, selected_indices, k, combination_weights,
           output_projections):
    del k
    B, S, k_in = selected_activations.shape
    n_process, n_input = combination_weights.shape
    d_model = output_projections.shape[1]
    PBLK = n_process // _PT
    SBLK = S // _ST

    idx_flat = selected_indices.reshape(-1).astype(jnp.int32)

    def fused_call(bc, a_b, wsel_b, proj, prev):
        scratch = [
            pltpu.VMEM((_PT, S, PBLK), jnp.float32),
            pltpu.VMEM((_PT, 1, PBLK), jnp.float32),
            pltpu.VMEM((_PT, 1, PBLK), jnp.float32),
        ]
        out_spec = pl.BlockSpec(
            (1, SBLK, d_model),
            lambda b, i: (bc, jnp.where(i < _PT, 0, i - _PT), 0))
        in_specs = [
            pl.BlockSpec((1, S, k_in), lambda b, i: (bc, 0, 0)),
            pl.BlockSpec((1, PBLK, k_in),
                         lambda b, i: (0, jnp.minimum(i, _PT - 1), 0)),
            pl.BlockSpec((n_process, d_model), lambda b, i: (0, 0)),
        ]
        args = [a_b, wsel_b, proj]
        aliases = {}
        if prev is not None:
            in_specs.append(pl.BlockSpec(memory_space=pl.ANY))
            args.append(prev)
            aliases = {3: 0}
        return pl.pallas_call(
            _fused_body,
            grid=(1, _PT + _ST),
            in_specs=in_specs,
            out_specs=out_spec,
            out_shape=jax.ShapeDtypeStruct((B, S, d_model), jnp.float32),
            scratch_shapes=scratch,
            input_output_aliases=aliases,
        )(*args)

    proj_bf16 = output_projections.astype(jnp.bfloat16)
    out = None
    for b in range(B):
        wsel_b = _sc_gather_cols(combination_weights, idx_flat, k_in,
                                 b * k_in)
        wsel_b = wsel_b.reshape(1, n_process, k_in)
        out = fused_call(b, selected_activations, wsel_b, proj_bf16, out)
    return out


# speedup vs baseline: 1.1617x; 1.1617x over previous
import jax
import jax.numpy as jnp
from jax import lax
from jax.experimental import pallas as pl
from jax.experimental.pallas import tpu as pltpu
from jax.experimental.pallas import tpu_sc as plsc

KSEL = 256

_NC, _NS = 2, 16
_NW = _NC * _NS

_ERF_ALPHA = (-2.72614225801306e-10, 2.77068142495902e-08,
              -2.10102402082508e-06, -5.69250639462346e-05,
              -7.34990630326855e-04, -2.95459980854025e-03,
              -1.60960333262415e-02)
_ERF_BETA = (-1.45660718464996e-05, -2.13374055278905e-04,
             -1.68282697438203e-03, -7.37332916720468e-03,
             -1.42647390514189e-02)


def _erf(z):
    z = jnp.clip(z, -4.0, 4.0)
    z2 = z * z
    alpha = jnp.float32(_ERF_ALPHA[0])
    for c in _ERF_ALPHA[1:]:
        alpha = alpha * z2 + jnp.float32(c)
    beta = jnp.float32(_ERF_BETA[0])
    for c in _ERF_BETA[1:]:
        beta = beta * z2 + jnp.float32(c)
    return z * alpha / beta


def _gelu_exact(x):
    return 0.5 * x * (1.0 + _erf(x * jnp.float32(0.7071067811865476)))


def _make_sc_colgather_body(off):
    def body(w_ref, idx_ref, out_ref, idx_v, row_a, row_b, out_v, sem_a,
             sem_b, osem):
        rows = out_v.shape[0]
        k_in = out_v.shape[1]
        wid = lax.axis_index("s") * _NC + lax.axis_index("c")
        p0 = wid * rows

        ck = row_a.shape[0]
        pltpu.sync_copy(idx_ref.at[pl.ds(off, k_in)], idx_v)
        pltpu.async_copy(w_ref.at[pl.ds(p0, ck)], row_a, sem_a)

        def extract(buf, r0):
            for g in range(k_in // 16):
                cvec = idx_v[pl.ds(g * 16, 16)]
                for rr in range(ck):
                    rvec = jnp.full((16,), rr, jnp.int32)
                    vals = plsc.load_gather(buf, [rvec, cvec])
                    out_v[r0 + rr, pl.ds(g * 16, 16)] = vals

        n_pairs = rows // (2 * ck)

        def pair_body(cp, carry):
            r0 = 2 * ck * cp
            pltpu.async_copy(w_ref.at[pl.ds(p0 + r0 + ck, ck)], row_b, sem_b)
            pltpu.make_async_copy(w_ref.at[pl.ds(p0, ck)], row_a, sem_a).wait()
            extract(row_a, r0)
            pltpu.async_copy(out_v.at[pl.ds(r0, ck)],
                             out_ref.at[pl.ds(p0 + r0, ck)], osem)

            @pl.when(cp + 1 < n_pairs)
            def _():
                pltpu.async_copy(w_ref.at[pl.ds(p0 + r0 + 2 * ck, ck)], row_a,
                                 sem_a)
            pltpu.make_async_copy(w_ref.at[pl.ds(p0, ck)], row_b, sem_b).wait()
            extract(row_b, r0 + ck)
            pltpu.async_copy(out_v.at[pl.ds(r0 + ck, ck)],
                             out_ref.at[pl.ds(p0 + r0 + ck, ck)], osem)
            return carry

        lax.fori_loop(0, n_pairs, pair_body, jnp.int32(0))
        for _ in range(2 * n_pairs):
            pltpu.make_async_copy(out_v.at[pl.ds(0, ck)],
                                  out_ref.at[pl.ds(p0, ck)], osem).wait()

    return body


def _sc_gather_cols(w, idx_full, k_in, off):
    n_process, n_input = w.shape
    rows = n_process // _NW
    mesh = plsc.VectorSubcoreMesh(core_axis_name="c", subcore_axis_name="s")
    return pl.kernel(
        _make_sc_colgather_body(off),
        out_type=jax.ShapeDtypeStruct((n_process, k_in), w.dtype),
        mesh=mesh,
        compiler_params=pltpu.CompilerParams(needs_layout_passes=False),
        scratch_types=[
            pltpu.VMEM((k_in,), jnp.int32),
            pltpu.VMEM((8, n_input), w.dtype),
            pltpu.VMEM((8, n_input), w.dtype),
            pltpu.VMEM((rows, k_in), w.dtype),
            pltpu.SemaphoreType.DMA,
            pltpu.SemaphoreType.DMA,
            pltpu.SemaphoreType.DMA,
        ],
    )(w, idx_full)


def _topk_mask(s):
    n = s.shape[1]
    si = jax.lax.bitcast_convert_type(s, jnp.int32)
    keys = jnp.where(si >= 0, si, si ^ jnp.int32(0x7FFFFFFF))
    sign = jnp.int32(-2147483648)

    def tbody(i, p):
        bit = jnp.left_shift(jnp.int32(1), jnp.int32(31) - i)
        cand = p | bit
        cnt = jnp.sum(jnp.where(keys >= (cand ^ sign), jnp.int32(1), jnp.int32(0)))
        return jnp.where(cnt >= KSEL, cand, p)

    p = jax.lax.fori_loop(0, 32, tbody, jnp.int32(0))
    thr = p ^ sign

    gt = keys > thr
    eq = keys == thr
    deficit = KSEL - jnp.sum(jnp.where(gt, jnp.int32(1), jnp.int32(0)))
    pidx = jax.lax.broadcasted_iota(jnp.int32, (1, n), 1)

    def mbody(i, lohi):
        lo, hi = lohi
        mid = (lo + hi) // 2
        cnt = jnp.sum(jnp.where(eq & (pidx <= mid), jnp.int32(1), jnp.int32(0)))
        ok = cnt >= deficit
        return jnp.where(ok, lo, mid + 1), jnp.where(ok, mid, hi)

    lo, _ = jax.lax.fori_loop(0, 11, mbody, (jnp.int32(0), jnp.int32(n - 1)))
    return (gt | (eq & (pidx <= lo))).astype(jnp.float32)


_PT = 8
_ST = 4


def _fused_body(a_ref, w_ref, p_ref, *rest):
    out_ref, pa_scr, sc_scr, mask_scr = rest[-4:]
    i = pl.program_id(1)
    pblk = pa_scr.shape[2]
    sblk = out_ref.shape[1]

    @pl.when(i < _PT)
    def _mm1():
        acts = lax.dot_general(a_ref[0], w_ref[0],
                               (((1,), (1,)), ((), ())),
                               preferred_element_type=jnp.float32)
        pa = _gelu_exact(acts)
        pa_scr[i] = pa
        sc_scr[i] = jnp.sum(pa, axis=0, keepdims=True)

    @pl.when(i == _PT)
    def _mask():
        s = jnp.concatenate([sc_scr[j] for j in range(_PT)], axis=-1)
        mask = _topk_mask(s)
        for j in range(_PT):
            mask_scr[j] = mask[:, j * pblk:(j + 1) * pblk]

    @pl.when(i >= _PT)
    def _mm2():
        st = i - _PT
        acc = jnp.zeros((sblk, out_ref.shape[2]), jnp.float32)
        for j in range(_PT):
            pa = pa_scr[j, pl.ds(st * sblk, sblk), :] * mask_scr[j]
            acc += jnp.dot(pa.astype(jnp.bfloat16),
                           p_ref[pl.ds(j * pblk, pblk), :],
                           preferred_element_type=jnp.float32)
        out_ref[0] = acc


def kernel(selected_activations, selected_indices, k, combination_weights,
           output_projections):
    del k
    B, S, k_in = selected_activations.shape
    n_process, n_input = combination_weights.shape
    d_model = output_projections.shape[1]
    PBLK = n_process // _PT
    SBLK = S // _ST

    idx_flat = selected_indices.reshape(-1).astype(jnp.int32)

    def fused_call(bc, a_b, wsel_b, proj, prev):
        scratch = [
            pltpu.VMEM((_PT, S, PBLK), jnp.float32),
            pltpu.VMEM((_PT, 1, PBLK), jnp.float32),
            pltpu.VMEM((_PT, 1, PBLK), jnp.float32),
        ]
        out_spec = pl.BlockSpec(
            (1, SBLK, d_model),
            lambda b, i: (bc, jnp.where(i < _PT, 0, i - _PT), 0))
        in_specs = [
            pl.BlockSpec((1, S, k_in), lambda b, i: (bc, 0, 0)),
            pl.BlockSpec((1, PBLK, k_in),
                         lambda b, i: (0, jnp.minimum(i, _PT - 1), 0)),
            pl.BlockSpec((n_process, d_model), lambda b, i: (0, 0)),
        ]
        args = [a_b, wsel_b, proj]
        aliases = {}
        if prev is not None:
            in_specs.append(pl.BlockSpec(memory_space=pl.ANY))
            args.append(prev)
            aliases = {3: 0}
        return pl.pallas_call(
            _fused_body,
            grid=(1, _PT + _ST),
            in_specs=in_specs,
            out_specs=out_spec,
            out_shape=jax.ShapeDtypeStruct((B, S, d_model), jnp.float32),
            scratch_shapes=scratch,
            input_output_aliases=aliases,
        )(*args)

    proj_bf16 = output_projections.astype(jnp.bfloat16)
    out = None
    for b in range(B):
        wsel_b = _sc_gather_cols(combination_weights, idx_flat, k_in,
                                 b * k_in)
        wsel_b = wsel_b.reshape(1, n_process, k_in)
        out = fused_call(b, selected_activations, wsel_b, proj_bf16, out)
    return out
